# baseline (speedup 1.0000x reference)
import functools

import jax
import jax.numpy as jnp
from jax import lax
from jax.experimental import pallas as pl
from jax.experimental.pallas import tpu as pltpu
from jax.experimental.pallas import tpu_sc as plsc

_SC_NC = 2
_SC_NS = 16
_SC_NW = _SC_NC * _SC_NS


def _sq_norm_kernel(emb_ref, se_ref, epad_ref):
    emb = emb_ref[...]
    emb2 = emb * emb
    ones = jnp.ones((1, emb2.shape[1]), jnp.float32)
    se_ref[...] = jax.lax.dot_general(
        ones, emb2, (((1,), (1,)), ((), ())),
        precision=jax.lax.Precision.HIGHEST,
        preferred_element_type=jnp.float32)
    d = emb.shape[1]
    epad_ref[:, :d] = emb
    epad_ref[:, d:] = jnp.zeros_like(emb)


def _vq_block_kernel(f_ref, w_ref, b_ref, g_ref, bb_ref, emb_ref, se_ref,
                     iota_ref, idx_ref, enc_ref):
    x = jnp.dot(f_ref[...], w_ref[...],
                preferred_element_type=jnp.float32) + b_ref[...]
    mu = jnp.mean(x, axis=-1, keepdims=True)
    var = jnp.mean((x - mu) ** 2, axis=-1, keepdims=True)
    xn = (x - mu) / jnp.sqrt(var + 1e-5) * g_ref[...] + bb_ref[...]

    emb = emb_ref[...]
    sf = jnp.sum(xn * xn, axis=1, keepdims=True)
    mm_n2 = jax.lax.dot_general(xn * (-2.0), emb, (((1,), (1,)), ((), ())),
                                preferred_element_type=jnp.float32)
    dist = (sf + se_ref[...]) + mm_n2

    k = emb.shape[0]
    minval = jnp.min(dist, axis=1, keepdims=True)
    iota = iota_ref[...]
    idx_f = jnp.min(jnp.where(dist == minval, iota, float(k)),
                    axis=1, keepdims=True)
    idx_ref[...] = idx_f.astype(jnp.int32)

    enc = (iota == idx_f).astype(jnp.float32)
    enc_ref[...] = enc


def _sc_gather(emb_pad, idx_flat, d):
    n = idx_flat.shape[0]
    k, d_pad = emb_pad.shape
    b_per_w = n // _SC_NW

    mesh = plsc.VectorSubcoreMesh(core_axis_name="c", subcore_axis_name="s")

    @functools.partial(
        pl.kernel, mesh=mesh,
        out_type=jax.ShapeDtypeStruct((n, d_pad), jnp.float32),
        scratch_types=[
            pltpu.VMEM((b_per_w,), jnp.int32),
            pltpu.VMEM((b_per_w, d_pad), jnp.float32),
            pltpu.SemaphoreType.DMA,
        ],
    )
    def gather_kernel(emb_hbm, idx_hbm, out_hbm, idx_v, rows_v, sem):
        wid = lax.axis_index("s") * _SC_NC + lax.axis_index("c")
        base = wid * b_per_w
        pltpu.sync_copy(idx_hbm.at[pl.ds(base, b_per_w)], idx_v)
        pltpu.async_copy(emb_hbm.at[idx_v], rows_v, sem).wait()
        pltpu.sync_copy(rows_v, out_hbm.at[pl.ds(base, b_per_w)])

    return gather_kernel(emb_pad, idx_flat)[:, :d]


@jax.jit
def kernel(features, proj_w, proj_b, ln_g, ln_b, emb):
    b_sz, t_sz, nhidden = features.shape
    n = b_sz * t_sz
    k, d = emb.shape
    flat_f = features.reshape(n, nhidden)

    se, emb_pad = pl.pallas_call(
        _sq_norm_kernel,
        out_shape=[
            jax.ShapeDtypeStruct((1, k), jnp.float32),
            jax.ShapeDtypeStruct((k, 128), jnp.float32),
        ],
    )(emb)

    blk = 128
    grid = (n // blk,)

    idx, enc = pl.pallas_call(
        _vq_block_kernel,
        grid=grid,
        in_specs=[
            pl.BlockSpec((blk, nhidden), lambda i: (i, 0)),
            pl.BlockSpec((nhidden, d), lambda i: (0, 0)),
            pl.BlockSpec((1, d), lambda i: (0, 0)),
            pl.BlockSpec((1, d), lambda i: (0, 0)),
            pl.BlockSpec((1, d), lambda i: (0, 0)),
            pl.BlockSpec((k, d), lambda i: (0, 0)),
            pl.BlockSpec((1, k), lambda i: (0, 0)),
            pl.BlockSpec((1, k), lambda i: (0, 0)),
        ],
        out_specs=[
            pl.BlockSpec((blk, 1), lambda i: (i, 0)),
            pl.BlockSpec((blk, k), lambda i: (i, 0)),
        ],
        out_shape=[
            jax.ShapeDtypeStruct((n, 1), jnp.int32),
            jax.ShapeDtypeStruct((n, k), jnp.float32),
        ],
        compiler_params=pltpu.CompilerParams(
            dimension_semantics=("parallel",)),
    )(flat_f, proj_w, proj_b.reshape(1, d), ln_g.reshape(1, d),
      ln_b.reshape(1, d), emb, se,
      jnp.arange(k, dtype=jnp.float32).reshape(1, k))

    q = _sc_gather(emb_pad, idx.reshape(n), d)
    return (q.reshape(b_sz, t_sz, d), idx, enc)


# speedup vs baseline: 1.0723x; 1.0723x over previous
import functools

import jax
import jax.numpy as jnp
from jax import lax
from jax.experimental import pallas as pl
from jax.experimental.pallas import tpu as pltpu
from jax.experimental.pallas import tpu_sc as plsc

_SC_NC = 2
_SC_NS = 16
_SC_NW = _SC_NC * _SC_NS


def _sq_norm_kernel(emb_ref, se_ref, epad_ref):
    emb = emb_ref[...]
    emb2 = emb * emb
    ones = jnp.ones((1, emb2.shape[1]), jnp.float32)
    se_ref[...] = jax.lax.dot_general(
        ones, emb2, (((1,), (1,)), ((), ())),
        precision=jax.lax.Precision.HIGHEST,
        preferred_element_type=jnp.float32)
    d = emb.shape[1]
    epad_ref[:, :d] = emb
    epad_ref[:, d:] = jnp.zeros_like(emb)


def _vq_block_kernel(f_ref, w_ref, b_ref, g_ref, bb_ref, emb_ref, se_ref,
                     iota_ref, idx_ref, enc_ref):
    x = jnp.dot(f_ref[...], w_ref[...],
                preferred_element_type=jnp.float32) + b_ref[...]
    mu = jnp.mean(x, axis=-1, keepdims=True)
    var = jnp.mean((x - mu) ** 2, axis=-1, keepdims=True)
    xn = (x - mu) / jnp.sqrt(var + 1e-5) * g_ref[...] + bb_ref[...]

    emb = emb_ref[...]
    sf = jnp.sum(xn * xn, axis=1, keepdims=True)
    mm_n2 = jax.lax.dot_general(xn * (-2.0), emb, (((1,), (1,)), ((), ())),
                                preferred_element_type=jnp.float32)
    dist = (sf + se_ref[...]) + mm_n2

    k = emb.shape[0]
    minval = jnp.min(dist, axis=1, keepdims=True)
    iota = iota_ref[...]
    idx_f = jnp.min(jnp.where(dist == minval, iota, float(k)),
                    axis=1, keepdims=True)
    idx_ref[...] = idx_f.astype(jnp.int32)

    enc = (iota == idx_f).astype(jnp.float32)
    enc_ref[...] = enc


def _sc_gather(emb_pad, idx_flat, d):
    n = idx_flat.shape[0]
    k, d_pad = emb_pad.shape
    b_per_w = n // _SC_NW

    mesh = plsc.VectorSubcoreMesh(core_axis_name="c", subcore_axis_name="s")

    @functools.partial(
        pl.kernel, mesh=mesh,
        out_type=jax.ShapeDtypeStruct((n, d_pad), jnp.float32),
        scratch_types=[
            pltpu.VMEM((b_per_w,), jnp.int32),
            pltpu.VMEM((b_per_w, d_pad), jnp.float32),
            pltpu.SemaphoreType.DMA,
        ],
    )
    def gather_kernel(emb_hbm, idx_hbm, out_hbm, idx_v, rows_v, sem):
        wid = lax.axis_index("s") * _SC_NC + lax.axis_index("c")
        base = wid * b_per_w
        pltpu.sync_copy(idx_hbm.at[pl.ds(base, b_per_w)], idx_v)
        pltpu.async_copy(emb_hbm.at[idx_v], rows_v, sem).wait()
        pltpu.sync_copy(rows_v, out_hbm.at[pl.ds(base, b_per_w)])

    return gather_kernel(emb_pad, idx_flat)[:, :d]


@jax.jit
def kernel(features, proj_w, proj_b, ln_g, ln_b, emb):
    b_sz, t_sz, nhidden = features.shape
    n = b_sz * t_sz
    k, d = emb.shape
    flat_f = features.reshape(n, nhidden)

    se, emb_pad = pl.pallas_call(
        _sq_norm_kernel,
        out_shape=[
            jax.ShapeDtypeStruct((1, k), jnp.float32),
            jax.ShapeDtypeStruct((k, 128), jnp.float32),
        ],
    )(emb)

    blk = 256
    grid = (n // blk,)

    idx, enc = pl.pallas_call(
        _vq_block_kernel,
        grid=grid,
        in_specs=[
            pl.BlockSpec((blk, nhidden), lambda i: (i, 0)),
            pl.BlockSpec((nhidden, d), lambda i: (0, 0)),
            pl.BlockSpec((1, d), lambda i: (0, 0)),
            pl.BlockSpec((1, d), lambda i: (0, 0)),
            pl.BlockSpec((1, d), lambda i: (0, 0)),
            pl.BlockSpec((k, d), lambda i: (0, 0)),
            pl.BlockSpec((1, k), lambda i: (0, 0)),
            pl.BlockSpec((1, k), lambda i: (0, 0)),
        ],
        out_specs=[
            pl.BlockSpec((blk, 1), lambda i: (i, 0)),
            pl.BlockSpec((blk, k), lambda i: (i, 0)),
        ],
        out_shape=[
            jax.ShapeDtypeStruct((n, 1), jnp.int32),
            jax.ShapeDtypeStruct((n, k), jnp.float32),
        ],
        compiler_params=pltpu.CompilerParams(
            dimension_semantics=("arbitrary",)),
    )(flat_f, proj_w, proj_b.reshape(1, d), ln_g.reshape(1, d),
      ln_b.reshape(1, d), emb, se,
      jnp.arange(k, dtype=jnp.float32).reshape(1, k))

    q = _sc_gather(emb_pad, idx.reshape(n), d)
    return (q.reshape(b_sz, t_sz, d), idx, enc)
